# baseline (speedup 1.0000x reference)
import functools

import jax
import jax.numpy as jnp
from jax import lax
from jax.experimental import pallas as pl
from jax.experimental.pallas import tpu as pltpu
from jax.experimental.pallas import tpu_sc as plsc

N = 10000
E = 320000
D = 128
Z = 64

NC = 2
NS = 16
NW = NC * NS
CH = 128
KCH = 80
E_PAD = NW * KCH * CH
PADR = 8
N2 = N + PADR
DPT = 624

_mesh = plsc.VectorSubcoreMesh(
    core_axis_name="c", subcore_axis_name="s", num_cores=NC, num_subcores=NS
)


def _deg_body(dstp_hbm, deg_hbm, deg_sh, dst_all, zv, ones_v, ones_t, dsem):
    cid = lax.axis_index("c")
    sid = lax.axis_index("s")
    wid = sid * NC + cid
    for i in range(DPT // 16):
        zv[pl.ds(i * 16, 16)] = jnp.zeros((16,), jnp.float32)
    for i in range(CH // 16):
        ones_v[pl.ds(i * 16, 16)] = jnp.ones((16,), jnp.float32)
    ones_t[...] = jnp.ones((16,), jnp.float32)
    pltpu.sync_copy(dstp_hbm.at[wid], dst_all)
    off = pl.multiple_of(sid * DPT, 8)
    pltpu.sync_copy(zv, deg_sh.at[pl.ds(off, DPT)])
    @pl.when(sid == NS - 1)
    def _():
        pltpu.sync_copy(zv.at[pl.ds(0, 16 + PADR)],
                        deg_sh.at[pl.ds(NS * DPT, 16 + PADR)])
    plsc.subcore_barrier()

    def chunk(k, carry):
        pltpu.async_copy(ones_v, deg_sh.at[dst_all.at[k]], dsem, add=True)
        return carry

    lax.fori_loop(0, KCH, chunk, 0)

    def chunk_drain(k, carry):
        pltpu.make_async_copy(ones_v, deg_sh.at[pl.ds(0, CH)], dsem).wait()
        return carry

    lax.fori_loop(0, KCH, chunk_drain, 0)
    plsc.subcore_barrier()
    hoff = pl.multiple_of(cid * N + sid * DPT, 8)
    pltpu.sync_copy(deg_sh.at[pl.ds(off, DPT)], zv)
    pltpu.sync_copy(zv, deg_hbm.at[pl.ds(hoff, DPT)])
    @pl.when(sid == NS - 1)
    def _():
        pltpu.sync_copy(deg_sh.at[pl.ds(NS * DPT, 16)], ones_t)
        pltpu.sync_copy(ones_t, deg_hbm.at[pl.ds(cid * N + NS * DPT, 16)])


_deg_call = pl.kernel(
    _deg_body,
    out_type=jax.ShapeDtypeStruct((NC * N,), jnp.float32),
    mesh=_mesh,
    scratch_types=[
        pltpu.VMEM_SHARED((N2,), jnp.float32),
        pltpu.VMEM((KCH, CH), jnp.int32),
        pltpu.VMEM((DPT,), jnp.float32),
        pltpu.VMEM((CH,), jnp.float32),
        pltpu.VMEM((16,), jnp.float32),
        pltpu.SemaphoreType.DMA,
    ],
)


_CP_SZ = (CH, CH, CH, CH, DPT - 4 * CH)


def _agg_body(g_hbm, srcp_hbm, dstp_hbm, zeros_hbm, part_hbm,
              acc_sh, dst_all, srcv0, srcv1, srcv2, srcv3, rows0, rows1,
              gsem, isem, ssem):
    cid = lax.axis_index("c")
    sid = lax.axis_index("s")
    wid = sid * NC + cid
    rows = [rows0, rows1]
    srcv = [srcv0, srcv1, srcv2, srcv3]

    def src_async(k, s):
        pltpu.async_copy(srcp_hbm.at[wid, k], srcv[s], isem)

    def src_wait(s):
        pltpu.make_async_copy(srcp_hbm.at[0, 0], srcv[s], isem).wait()

    def gather_start(s, b):
        pltpu.async_copy(g_hbm.at[srcv[s]], rows[b], gsem)

    def gather_wait(b):
        pltpu.make_async_copy(g_hbm.at[pl.ds(0, CH)], rows[b], gsem).wait()

    for k0 in range(1, 4):
        src_async(k0, k0)
    pltpu.sync_copy(zeros_hbm, rows1)
    pltpu.sync_copy(dstp_hbm.at[wid], dst_all)
    off = 0
    for sz in _CP_SZ:
        pltpu.sync_copy(rows1.at[pl.ds(0, sz)],
                        acc_sh.at[pl.ds(sid * DPT + off, sz)])
        off += sz
    @pl.when(sid == NS - 1)
    def _():
        pltpu.sync_copy(rows1.at[pl.ds(0, 16 + PADR)],
                        acc_sh.at[pl.ds(NS * DPT, 16 + PADR)])
    pltpu.sync_copy(srcp_hbm.at[wid, 0], srcv0)
    plsc.subcore_barrier()
    gather_start(0, 0)

    def scatter_drain():
        pltpu.make_async_copy(rows0, acc_sh.at[pl.ds(0, CH)], ssem).wait()

    def ring(j, carry):
        for u in range(4):
            k = j * 4 + u
            b = u % 2
            @pl.when(k + 1 < KCH)
            def _():
                src_wait((u + 1) % 4)
                @pl.when(k >= 1)
                def _():
                    scatter_drain()
                gather_start((u + 1) % 4, 1 - b)
            gather_wait(b)
            pltpu.async_copy(rows[b], acc_sh.at[dst_all.at[k]], ssem, add=True)
            @pl.when(k + 4 < KCH)
            def _():
                src_async(k + 4, u)
        return carry

    lax.fori_loop(0, KCH // 4, ring, 0)
    scatter_drain()
    scatter_drain()
    plsc.subcore_barrier()
    def wr_drain(sz):
        pltpu.make_async_copy(rows0.at[pl.ds(0, sz)],
                              part_hbm.at[0, pl.ds(0, sz)], isem).wait()

    off = 0
    for i, sz in enumerate(_CP_SZ):
        if i >= 2:
            wr_drain(_CP_SZ[i - 2])
        r0 = pl.multiple_of(sid * DPT + off, 8)
        pltpu.sync_copy(acc_sh.at[pl.ds(r0, sz)], rows[i % 2].at[pl.ds(0, sz)])
        pltpu.async_copy(rows[i % 2].at[pl.ds(0, sz)],
                         part_hbm.at[cid, pl.ds(r0, sz)], isem)
        off += sz
    wr_drain(_CP_SZ[3])
    wr_drain(_CP_SZ[4])
    @pl.when(sid == NS - 1)
    def _():
        pltpu.sync_copy(acc_sh.at[pl.ds(NS * DPT, 16)], rows0.at[pl.ds(0, 16)])
        pltpu.sync_copy(rows0.at[pl.ds(0, 16)], part_hbm.at[cid, pl.ds(NS * DPT, 16)])


_agg_call = pl.kernel(
    _agg_body,
    out_type=jax.ShapeDtypeStruct((NC, N, D), jnp.float32),
    mesh=_mesh,
    scratch_types=[
        pltpu.VMEM_SHARED((N2, D), jnp.float32),
        pltpu.VMEM((KCH, CH), jnp.int32),
        pltpu.VMEM((CH,), jnp.int32),
        pltpu.VMEM((CH,), jnp.int32),
        pltpu.VMEM((CH,), jnp.int32),
        pltpu.VMEM((CH,), jnp.int32),
        pltpu.VMEM((CH, D), jnp.float32),
        pltpu.VMEM((CH, D), jnp.float32),
        pltpu.SemaphoreType.DMA,
        pltpu.SemaphoreType.DMA,
        pltpu.SemaphoreType.DMA,
    ],
)


BM = 10000
GRID = N // BM


def _sinv(deg_blk):
    return lax.rsqrt(jnp.maximum(deg_blk[0] + deg_blk[1], 1.0))


def _tc1_body(x_ref, deg_ref, w1_ref, b1_ref, o_ref):
    xb = x_ref[...]
    nrm = jnp.sqrt(jnp.sum(xb * xb, axis=1, keepdims=True))
    xn = xb / jnp.maximum(nrm, 1e-12)
    s = _sinv(deg_ref[...])
    o_ref[...] = (jnp.dot(xn, w1_ref[...],
                          preferred_element_type=jnp.float32) + b1_ref[...]) * s


def _tc2_body(p_ref, deg_ref, w2_ref, b2_ref, o_ref):
    p = p_ref[...]
    s = _sinv(deg_ref[...])
    h = jax.nn.relu((p[0] + p[1]) * s)
    o_ref[...] = (jnp.dot(h, w2_ref[...],
                          preferred_element_type=jnp.float32) + b2_ref[...]) * s


def _tc3_body(p_ref, deg_ref, y_ref, wy_ref, by_ref, woh_ref, woy_ref, bo_ref,
              wz_ref, bz_ref, mu_ref, lv_ref):
    p = p_ref[...]
    s = _sinv(deg_ref[...])
    h2 = (p[0] + p[1]) * s
    ye = jnp.dot(y_ref[...], wy_ref[...],
                 preferred_element_type=jnp.float32) + by_ref[...]
    out = (jnp.dot(h2, woh_ref[...], preferred_element_type=jnp.float32)
           + jnp.dot(ye, woy_ref[...], preferred_element_type=jnp.float32)
           + bo_ref[...])
    z = jnp.dot(out, wz_ref[...], preferred_element_type=jnp.float32) + bz_ref[...]
    e = jnp.where(z > 0, z, jnp.exp(jnp.minimum(z, 0.0)) - 1.0)
    mu_ref[...] = e[:, :Z]
    lv_ref[...] = e[:, Z:]


def _row_spec(shape_tail):
    return pl.BlockSpec((BM,) + shape_tail, lambda i: (i,) + (0,) * len(shape_tail))


_deg_spec = pl.BlockSpec((NC, BM, 1), lambda i: (0, i, 0))
_w_spec = pl.BlockSpec((D, D), lambda i: (0, 0))
_b_spec = pl.BlockSpec((1, D), lambda i: (0, 0))
_p_spec = pl.BlockSpec((NC, BM, D), lambda i: (0, i, 0))
_o_shape = jax.ShapeDtypeStruct((N, D), jnp.float32)
_o_spec = _row_spec((D,))

_tc1_call = pl.pallas_call(
    _tc1_body,
    grid=(GRID,),
    in_specs=[_row_spec((D,)), _deg_spec, _w_spec, _b_spec],
    out_specs=_o_spec,
    out_shape=_o_shape,
)

_tc2_call = pl.pallas_call(
    _tc2_body,
    grid=(GRID,),
    in_specs=[_p_spec, _deg_spec, _w_spec, _b_spec],
    out_specs=_o_spec,
    out_shape=_o_shape,
)

_tc3_call = pl.pallas_call(
    _tc3_body,
    grid=(GRID,),
    in_specs=[_p_spec, _deg_spec, _row_spec((8,)),
              pl.BlockSpec((8, D), lambda i: (0, 0)), _b_spec,
              _w_spec, _w_spec, _b_spec, _w_spec, _b_spec],
    out_specs=[_row_spec((Z,)), _row_spec((Z,))],
    out_shape=[jax.ShapeDtypeStruct((N, Z), jnp.float32),
               jax.ShapeDtypeStruct((N, Z), jnp.float32)],
)


@jax.jit
def kernel(x, adj, y, W1, b1, W2, b2, Wy, by, Wo, bo, Wmu, bmu, Wlv, blv):
    pad = E_PAD - E
    pi = jnp.arange(pad, dtype=jnp.int32)
    srcp = jnp.concatenate([adj[0], (pi * 13) % N]).reshape(NW, KCH, CH)
    dstp = jnp.concatenate([adj[1], N + (pi % PADR)]).reshape(NW, KCH, CH)
    zeros_rows = jnp.zeros((CH, D), jnp.float32)

    degp = _deg_call(dstp)
    deg3 = degp.reshape(NC, N, 1)

    g1 = _tc1_call(x, deg3, W1, b1.reshape(1, D))
    p1 = _agg_call(g1, srcp, dstp, zeros_rows)
    g2 = _tc2_call(p1, deg3, W2, b2.reshape(1, D))
    p2 = _agg_call(g2, srcp, dstp, zeros_rows)

    y8 = jnp.concatenate([y, jnp.zeros((N, 1), jnp.float32)], axis=1)
    wy8 = jnp.concatenate([Wy, jnp.zeros((1, D), jnp.float32)], axis=0)
    wz = jnp.concatenate([Wmu, Wlv], axis=1)
    bz = jnp.concatenate([bmu, blv]).reshape(1, D)
    mu, logvar = _tc3_call(p2, deg3, y8, wy8, by.reshape(1, D),
                           Wo[:D], Wo[D:], bo.reshape(1, D), wz, bz)
    return mu, logvar


# speedup vs baseline: 1.0181x; 1.0181x over previous
import jax
import jax.numpy as jnp
from jax import lax
from jax.experimental import pallas as pl
from jax.experimental.pallas import tpu as pltpu
from jax.experimental.pallas import tpu_sc as plsc

N = 10000
E = 320000
D = 128
Z = 64

NC = 2
NS = 16
NW = NC * NS
CH = 128
KCH = 80
E_PAD = NW * KCH * CH
PADR = 8
N2 = N + PADR
DPT = 624

_mesh = plsc.VectorSubcoreMesh(
    core_axis_name="c", subcore_axis_name="s", num_cores=NC, num_subcores=NS
)


def _deg_body(dstp_hbm, deg_hbm, deg_sh, dst_all, zv, ones_v, ones_t, dsem):
    cid = lax.axis_index("c")
    sid = lax.axis_index("s")
    wid = sid * NC + cid
    for i in range(DPT // 16):
        zv[pl.ds(i * 16, 16)] = jnp.zeros((16,), jnp.float32)
    for i in range(CH // 16):
        ones_v[pl.ds(i * 16, 16)] = jnp.ones((16,), jnp.float32)
    ones_t[...] = jnp.ones((16,), jnp.float32)
    pltpu.sync_copy(dstp_hbm.at[wid], dst_all)
    off = pl.multiple_of(sid * DPT, 8)
    pltpu.sync_copy(zv, deg_sh.at[pl.ds(off, DPT)])
    @pl.when(sid == NS - 1)
    def _():
        pltpu.sync_copy(zv.at[pl.ds(0, 16 + PADR)],
                        deg_sh.at[pl.ds(NS * DPT, 16 + PADR)])
    plsc.subcore_barrier()

    def chunk(k, carry):
        pltpu.async_copy(ones_v, deg_sh.at[dst_all.at[k]], dsem, add=True)
        return carry

    lax.fori_loop(0, KCH, chunk, 0)

    def chunk_drain(k, carry):
        pltpu.make_async_copy(ones_v, deg_sh.at[pl.ds(0, CH)], dsem).wait()
        return carry

    lax.fori_loop(0, KCH, chunk_drain, 0)
    plsc.subcore_barrier()
    hoff = pl.multiple_of(cid * N + sid * DPT, 8)
    pltpu.sync_copy(deg_sh.at[pl.ds(off, DPT)], zv)
    pltpu.sync_copy(zv, deg_hbm.at[pl.ds(hoff, DPT)])
    @pl.when(sid == NS - 1)
    def _():
        pltpu.sync_copy(deg_sh.at[pl.ds(NS * DPT, 16)], ones_t)
        pltpu.sync_copy(ones_t, deg_hbm.at[pl.ds(cid * N + NS * DPT, 16)])


_deg_call = pl.kernel(
    _deg_body,
    out_type=jax.ShapeDtypeStruct((NC * N,), jnp.float32),
    mesh=_mesh,
    scratch_types=[
        pltpu.VMEM_SHARED((N2,), jnp.float32),
        pltpu.VMEM((KCH, CH), jnp.int32),
        pltpu.VMEM((DPT,), jnp.float32),
        pltpu.VMEM((CH,), jnp.float32),
        pltpu.VMEM((16,), jnp.float32),
        pltpu.SemaphoreType.DMA,
    ],
)


_CP_SZ = (CH, CH, CH, CH, DPT - 4 * CH)


def _agg_body(g_hbm, srcp_hbm, dstp_hbm, zeros_hbm, part_hbm,
              acc_sh, dst_all, srcv0, srcv1, srcv2, srcv3, rows0, rows1,
              gsem, isem, ssem):
    cid = lax.axis_index("c")
    sid = lax.axis_index("s")
    wid = sid * NC + cid
    rows = [rows0, rows1]
    srcv = [srcv0, srcv1, srcv2, srcv3]

    def src_async(k, s):
        pltpu.async_copy(srcp_hbm.at[wid, k], srcv[s], isem)

    def src_wait(s):
        pltpu.make_async_copy(srcp_hbm.at[0, 0], srcv[s], isem).wait()

    def gather_start(s, b):
        pltpu.async_copy(g_hbm.at[srcv[s]], rows[b], gsem)

    def gather_wait(b):
        pltpu.make_async_copy(g_hbm.at[pl.ds(0, CH)], rows[b], gsem).wait()

    for k0 in range(1, 4):
        src_async(k0, k0)
    pltpu.sync_copy(zeros_hbm, rows1)
    pltpu.sync_copy(dstp_hbm.at[wid], dst_all)
    off = 0
    for sz in _CP_SZ:
        pltpu.sync_copy(rows1.at[pl.ds(0, sz)],
                        acc_sh.at[pl.ds(sid * DPT + off, sz)])
        off += sz
    @pl.when(sid == NS - 1)
    def _():
        pltpu.sync_copy(rows1.at[pl.ds(0, 16 + PADR)],
                        acc_sh.at[pl.ds(NS * DPT, 16 + PADR)])
    pltpu.sync_copy(srcp_hbm.at[wid, 0], srcv0)
    plsc.subcore_barrier()
    gather_start(0, 0)

    def scatter_drain():
        pltpu.make_async_copy(rows0, acc_sh.at[pl.ds(0, CH)], ssem).wait()

    def ring(j, carry):
        for u in range(4):
            k = j * 4 + u
            b = u % 2
            @pl.when(k + 1 < KCH)
            def _():
                src_wait((u + 1) % 4)
                @pl.when(k >= 1)
                def _():
                    scatter_drain()
                gather_start((u + 1) % 4, 1 - b)
            gather_wait(b)
            pltpu.async_copy(rows[b], acc_sh.at[dst_all.at[k]], ssem, add=True)
            @pl.when(k + 4 < KCH)
            def _():
                src_async(k + 4, u)
        return carry

    lax.fori_loop(0, KCH // 4, ring, 0)
    scatter_drain()
    scatter_drain()
    plsc.subcore_barrier()
    def wr_drain(sz):
        pltpu.make_async_copy(rows0.at[pl.ds(0, sz)],
                              part_hbm.at[0, pl.ds(0, sz)], isem).wait()

    off = 0
    for i, sz in enumerate(_CP_SZ):
        if i >= 2:
            wr_drain(_CP_SZ[i - 2])
        r0 = pl.multiple_of(sid * DPT + off, 8)
        pltpu.sync_copy(acc_sh.at[pl.ds(r0, sz)], rows[i % 2].at[pl.ds(0, sz)])
        pltpu.async_copy(rows[i % 2].at[pl.ds(0, sz)],
                         part_hbm.at[cid, pl.ds(r0, sz)], isem)
        off += sz
    wr_drain(_CP_SZ[3])
    wr_drain(_CP_SZ[4])
    @pl.when(sid == NS - 1)
    def _():
        pltpu.sync_copy(acc_sh.at[pl.ds(NS * DPT, 16)], rows0.at[pl.ds(0, 16)])
        pltpu.sync_copy(rows0.at[pl.ds(0, 16)], part_hbm.at[cid, pl.ds(NS * DPT, 16)])


_agg_call = pl.kernel(
    _agg_body,
    out_type=jax.ShapeDtypeStruct((NC, N, D), jnp.float32),
    mesh=_mesh,
    scratch_types=[
        pltpu.VMEM_SHARED((N2, D), jnp.float32),
        pltpu.VMEM((KCH, CH), jnp.int32),
        pltpu.VMEM((CH,), jnp.int32),
        pltpu.VMEM((CH,), jnp.int32),
        pltpu.VMEM((CH,), jnp.int32),
        pltpu.VMEM((CH,), jnp.int32),
        pltpu.VMEM((CH, D), jnp.float32),
        pltpu.VMEM((CH, D), jnp.float32),
        pltpu.SemaphoreType.DMA,
        pltpu.SemaphoreType.DMA,
        pltpu.SemaphoreType.DMA,
    ],
)


BM = 5000
GRID = N // BM


def _sinv(deg_blk):
    return lax.rsqrt(jnp.maximum(deg_blk[0] + deg_blk[1], 1.0))


def _tc1_body(x_ref, deg_ref, w1_ref, b1_ref, o_ref):
    xb = x_ref[...]
    nrm = jnp.sqrt(jnp.sum(xb * xb, axis=1, keepdims=True))
    xn = xb / jnp.maximum(nrm, 1e-12)
    s = _sinv(deg_ref[...])
    o_ref[...] = (jnp.dot(xn, w1_ref[...],
                          preferred_element_type=jnp.float32) + b1_ref[...]) * s


def _tc2_body(p_ref, deg_ref, w2_ref, b2_ref, o_ref):
    p = p_ref[...]
    s = _sinv(deg_ref[...])
    h = jax.nn.relu((p[0] + p[1]) * s)
    o_ref[...] = (jnp.dot(h, w2_ref[...],
                          preferred_element_type=jnp.float32) + b2_ref[...]) * s


def _tc3_body(p_ref, deg_ref, y_ref, wy_ref, by_ref, woh_ref, woy_ref, bo_ref,
              wz_ref, bz_ref, mu_ref, lv_ref):
    p = p_ref[...]
    s = _sinv(deg_ref[...])
    h2 = (p[0] + p[1]) * s
    ye = jnp.dot(y_ref[...], wy_ref[...],
                 preferred_element_type=jnp.float32) + by_ref[...]
    out = (jnp.dot(h2, woh_ref[...], preferred_element_type=jnp.float32)
           + jnp.dot(ye, woy_ref[...], preferred_element_type=jnp.float32)
           + bo_ref[...])
    z = jnp.dot(out, wz_ref[...], preferred_element_type=jnp.float32) + bz_ref[...]
    e = jnp.where(z > 0, z, jnp.exp(jnp.minimum(z, 0.0)) - 1.0)
    mu_ref[...] = e[:, :Z]
    lv_ref[...] = e[:, Z:]


def _row_spec(shape_tail):
    return pl.BlockSpec((BM,) + shape_tail, lambda i: (i,) + (0,) * len(shape_tail))


_deg_spec = pl.BlockSpec((NC, BM, 1), lambda i: (0, i, 0))
_w_spec = pl.BlockSpec((D, D), lambda i: (0, 0))
_b_spec = pl.BlockSpec((1, D), lambda i: (0, 0))
_p_spec = pl.BlockSpec((NC, BM, D), lambda i: (0, i, 0))
_o_shape = jax.ShapeDtypeStruct((N, D), jnp.float32)
_o_spec = _row_spec((D,))

_tc1_call = pl.pallas_call(
    _tc1_body,
    grid=(GRID,),
    in_specs=[_row_spec((D,)), _deg_spec, _w_spec, _b_spec],
    out_specs=_o_spec,
    out_shape=_o_shape,
)

_tc2_call = pl.pallas_call(
    _tc2_body,
    grid=(GRID,),
    in_specs=[_p_spec, _deg_spec, _w_spec, _b_spec],
    out_specs=_o_spec,
    out_shape=_o_shape,
)

_tc3_call = pl.pallas_call(
    _tc3_body,
    grid=(GRID,),
    in_specs=[_p_spec, _deg_spec, _row_spec((8,)),
              pl.BlockSpec((8, D), lambda i: (0, 0)), _b_spec,
              _w_spec, _w_spec, _b_spec, _w_spec, _b_spec],
    out_specs=[_row_spec((Z,)), _row_spec((Z,))],
    out_shape=[jax.ShapeDtypeStruct((N, Z), jnp.float32),
               jax.ShapeDtypeStruct((N, Z), jnp.float32)],
)


@jax.jit
def kernel(x, adj, y, W1, b1, W2, b2, Wy, by, Wo, bo, Wmu, bmu, Wlv, blv):
    pad = E_PAD - E
    pi = jnp.arange(pad, dtype=jnp.int32)
    srcp = jnp.concatenate([adj[0], (pi * 13) % N]).reshape(NW, KCH, CH)
    dstp = jnp.concatenate([adj[1], N + (pi % PADR)]).reshape(NW, KCH, CH)
    zeros_rows = jnp.zeros((CH, D), jnp.float32)

    degp = _deg_call(dstp)
    deg3 = degp.reshape(NC, N, 1)

    g1 = _tc1_call(x, deg3, W1, b1.reshape(1, D))
    p1 = _agg_call(g1, srcp, dstp, zeros_rows)
    g2 = _tc2_call(p1, deg3, W2, b2.reshape(1, D))
    p2 = _agg_call(g2, srcp, dstp, zeros_rows)

    y8 = jnp.concatenate([y, jnp.zeros((N, 1), jnp.float32)], axis=1)
    wy8 = jnp.concatenate([Wy, jnp.zeros((1, D), jnp.float32)], axis=0)
    wz = jnp.concatenate([Wmu, Wlv], axis=1)
    bz = jnp.concatenate([bmu, blv]).reshape(1, D)
    mu, logvar = _tc3_call(p2, deg3, y8, wy8, by.reshape(1, D),
                           Wo[:D], Wo[D:], bo.reshape(1, D), wz, bz)
    return mu, logvar
